# baseline (speedup 1.0000x reference)
import functools

import jax
import jax.numpy as jnp
from jax import lax
from jax.experimental import pallas as pl
from jax.experimental.pallas import tpu as pltpu
from jax.experimental.pallas import tpu_sc as plsc

_NBUF = 5


@functools.lru_cache(maxsize=None)
def _build_gather(N, V, D, NC, NS, C):
    NW = NC * NS
    b_per_w = N // NW
    n_chunks = b_per_w // C
    n_grp = n_chunks // _NBUF

    mesh = plsc.VectorSubcoreMesh(core_axis_name="c", subcore_axis_name="s")

    @functools.partial(
        pl.kernel,
        mesh=mesh,
        out_type=jax.ShapeDtypeStruct((NW, n_chunks, C, D), jnp.float32),
        scratch_types=[
            pltpu.VMEM((n_chunks, C), jnp.int32),
            pltpu.VMEM((_NBUF, C, D), jnp.float32),
            pltpu.VMEM_SHARED((V, D), jnp.float32),
        ]
        + [pltpu.SemaphoreType.DMA] * (2 * _NBUF),
    )
    def k(table_hbm, idx_hbm, out_hbm, idx_v, rows_v, table_sh, *sems):
        gsems = sems[:_NBUF]
        ssems = sems[_NBUF:]
        cid = lax.axis_index("c")
        sid = lax.axis_index("s")
        wid = sid * NC + cid

        @pl.when(sid == 0)
        def _load_table():
            pltpu.sync_copy(table_hbm, table_sh)

        pltpu.sync_copy(idx_hbm.at[wid], idx_v)
        plsc.subcore_barrier()

        for b in range(_NBUF):
            pltpu.async_copy(table_sh.at[idx_v.at[b]], rows_v.at[b], gsems[b])

        def grp(g, carry):
            for b in range(_NBUF):
                j = _NBUF * g + b
                pltpu.make_async_copy(
                    table_sh.at[idx_v.at[j]], rows_v.at[b], gsems[b]
                ).wait()
                pltpu.async_copy(rows_v.at[b], out_hbm.at[wid, j], ssems[b])
                pltpu.make_async_copy(
                    rows_v.at[b], out_hbm.at[wid, j], ssems[b]
                ).wait()
                pltpu.async_copy(
                    table_sh.at[idx_v.at[j + _NBUF]], rows_v.at[b], gsems[b]
                )
            return carry

        lax.fori_loop(0, n_grp - 1, grp, 0)

        for b in range(_NBUF):
            j = _NBUF * (n_grp - 1) + b
            pltpu.make_async_copy(
                table_sh.at[idx_v.at[j]], rows_v.at[b], gsems[b]
            ).wait()
            pltpu.async_copy(rows_v.at[b], out_hbm.at[wid, j], ssems[b])
        for b in range(_NBUF):
            j = _NBUF * (n_grp - 1) + b
            pltpu.make_async_copy(rows_v.at[b], out_hbm.at[wid, j], ssems[b]).wait()

    return k


def kernel(x, doy_table, month_table):
    B, H = x.shape
    V, D = doy_table.shape
    N = B * H
    info = plsc.get_sparse_core_info()
    NC, NS = info.num_cores, info.num_subcores
    NW = NC * NS
    C = 128
    xw = x.reshape(NW, (N // NW) // C, C).astype(jnp.int32)
    out = _build_gather(N, V, D, NC, NS, C)(doy_table, xw)
    return out.reshape(B, H, D)


# speedup vs baseline: 1.0048x; 1.0048x over previous
import functools

import jax
import jax.numpy as jnp
from jax import lax
from jax.experimental import pallas as pl
from jax.experimental.pallas import tpu as pltpu
from jax.experimental.pallas import tpu_sc as plsc

_NBUF = 4


@functools.lru_cache(maxsize=None)
def _build_gather(N, V, D, NC, NS, C):
    NW = NC * NS
    b_per_w = N // NW
    n_chunks = b_per_w // C
    n_grp = n_chunks // _NBUF

    stripe = V // NS

    mesh = plsc.VectorSubcoreMesh(core_axis_name="c", subcore_axis_name="s")

    @functools.partial(
        pl.kernel,
        mesh=mesh,
        out_type=jax.ShapeDtypeStruct((NW, n_chunks, C, D), jnp.float32),
        scratch_types=[
            pltpu.VMEM((n_chunks, C), jnp.int32),
            pltpu.VMEM((_NBUF, C, D), jnp.float32),
            pltpu.VMEM_SHARED((V, D), jnp.float32),
        ]
        + [pltpu.SemaphoreType.DMA] * (2 * _NBUF),
    )
    def k(table_hbm, idx_hbm, out_hbm, idx_v, rows_v, table_sh, *sems):
        gsems = sems[:_NBUF]
        ssems = sems[_NBUF:]
        cid = lax.axis_index("c")
        sid = lax.axis_index("s")
        wid = sid * NC + cid

        off = sid * stripe
        pltpu.sync_copy(table_hbm.at[pl.ds(off, stripe)], table_sh.at[pl.ds(off, stripe)])

        head = 8
        pltpu.sync_copy(idx_hbm.at[wid, pl.ds(0, head)], idx_v.at[pl.ds(0, head)])
        plsc.subcore_barrier()

        for b in range(_NBUF):
            pltpu.async_copy(table_sh.at[idx_v.at[b]], rows_v.at[b], gsems[b])

        pltpu.sync_copy(
            idx_hbm.at[wid, pl.ds(head, n_chunks - head)],
            idx_v.at[pl.ds(head, n_chunks - head)],
        )

        def grp(g, carry):
            for b in range(_NBUF):
                j = _NBUF * g + b
                pltpu.make_async_copy(
                    table_sh.at[idx_v.at[j]], rows_v.at[b], gsems[b]
                ).wait()
                pltpu.async_copy(rows_v.at[b], out_hbm.at[wid, j], ssems[b])
                pltpu.make_async_copy(
                    rows_v.at[b], out_hbm.at[wid, j], ssems[b]
                ).wait()
                pltpu.async_copy(
                    table_sh.at[idx_v.at[j + _NBUF]], rows_v.at[b], gsems[b]
                )
            return carry

        lax.fori_loop(0, n_grp - 1, grp, 0)

        for b in range(_NBUF):
            j = _NBUF * (n_grp - 1) + b
            pltpu.make_async_copy(
                table_sh.at[idx_v.at[j]], rows_v.at[b], gsems[b]
            ).wait()
            pltpu.async_copy(rows_v.at[b], out_hbm.at[wid, j], ssems[b])
        for b in range(_NBUF):
            j = _NBUF * (n_grp - 1) + b
            pltpu.make_async_copy(rows_v.at[b], out_hbm.at[wid, j], ssems[b]).wait()

    return k


def kernel(x, doy_table, month_table):
    B, H = x.shape
    V, D = doy_table.shape
    N = B * H
    info = plsc.get_sparse_core_info()
    NC, NS = info.num_cores, info.num_subcores
    NW = NC * NS
    C = 128
    Vp = -(-V // (8 * NS)) * (8 * NS)
    table_p = jnp.pad(doy_table, ((0, Vp - V), (0, 0)))
    xw = x.reshape(NW, (N // NW) // C, C).astype(jnp.int32)
    out = _build_gather(N, Vp, D, NC, NS, C)(table_p, xw)
    return out.reshape(B, H, D)
